# baseline (speedup 1.0000x reference)
import functools

import jax
import jax.numpy as jnp
from jax import lax
from jax.experimental import pallas as pl
from jax.experimental.pallas import tpu as pltpu
from jax.experimental.pallas import tpu_sc as plsc

D = 128
CH = 80
SEG = 16
LN = 16
NCORE = 2
NSUB = 16
NW = NCORE * NSUB
BN = 1024


def _fcnn_block(v, W1, b1, W2, b2):
    h = lax.dot_general(v, W1, (((1,), (1,)), ((), ())),
                        preferred_element_type=jnp.float32)
    h = jnp.maximum(h + b1, 0.0)
    o = lax.dot_general(h, W2, (((1,), (1,)), ((), ())),
                        preferred_element_type=jnp.float32)
    return o + b2


def _dense_body(x_ref, gf_ref, W1a_ref, b1a_ref, W2a_ref, b2a_ref,
                W1b_ref, b1b_ref, W2b_ref, b2b_ref,
                W1c_ref, b1c_ref, W2c_ref, b2c_ref,
                h1_ref, h2_ref, g_ref):
    x = x_ref[...]
    h1_ref[...] = _fcnn_block(x, W1a_ref[...], b1a_ref[...],
                              W2a_ref[...], b2a_ref[...])
    h2_ref[...] = _fcnn_block(x, W1b_ref[...], b1b_ref[...],
                              W2b_ref[...], b2b_ref[...])

    @pl.when(pl.program_id(0) == 0)
    def _():
        g_ref[...] = _fcnn_block(gf_ref[...], W1c_ref[...], b1c_ref[...],
                                 W2c_ref[...], b2c_ref[...])


def _combine_body(x_ref, h1_ref, a0_ref, a1_ref, d_ref, g_ref,
                  out_ref):
    msg = a0_ref[0] + a1_ref[0]
    bn = msg.shape[0]
    nr = bn // D
    dsum = jnp.sum(d_ref[...], axis=0)
    rowsel = (lax.broadcasted_iota(jnp.int32, (bn, nr), 0) // D
              == lax.broadcasted_iota(jnp.int32, (bn, nr), 1))
    dfull = lax.dot_general(rowsel.astype(jnp.float32), dsum,
                            (((1,), (0,)), ((), ())),
                            preferred_element_type=jnp.float32)
    lanesel = (lax.broadcasted_iota(jnp.int32, (bn, D), 0) % D
               == lax.broadcasted_iota(jnp.int32, (bn, D), 1))
    den = jnp.sum(jnp.where(lanesel, dfull, 0.0), axis=1,
                  keepdims=True) + jnp.float32(1e-7)
    inter = h1_ref[...] + msg / den + g_ref[...]
    m = jnp.mean(inter, axis=1, keepdims=True)
    cen = inter - m
    var = jnp.mean(cen * cen, axis=1, keepdims=True)
    normed = cen * lax.rsqrt(var + jnp.float32(1e-5))
    out_ref[...] = x_ref[...] + jnp.maximum(normed, 0.0)


def _lane_gather(vec, idx):
    return lax.gather(
        vec, idx[:, None],
        lax.GatherDimensionNumbers(offset_dims=(), collapsed_slice_dims=(0,),
                                   start_index_map=(0,)),
        (1,), mode=lax.GatherScatterMode.PROMISE_IN_BOUNDS)


def _edge_body(npad, cpt0, cpt1, h2_hbm, tgt_hbm, oth_hbm, ef_hbm,
               msg_hbm, den_hbm,
               tgt_s, oth_s, ef_s, rows0_v, rows1_v, tlast_v, dacc_v, idx80_v,
               acc_v, dsp_v, g0, g1, s0, s1):
    c = lax.axis_index("c")
    s = lax.axis_index("s")
    rps = npad // NSUB
    zb = 64
    nseg = jnp.where(c == 0, cpt0 // SEG, cpt1 // SEG)
    tbase = jnp.where(c == 0, s * cpt0, NSUB * cpt0 + s * cpt1)
    iota = lax.iota(jnp.int32, LN)

    def _zero_row(i, _):
        for cc in range(D // LN):
            rows0_v[i, pl.ds(cc * LN, LN)] = jnp.zeros((LN,), jnp.float32)
            rows1_v[i, pl.ds(cc * LN, LN)] = jnp.zeros((LN,), jnp.float32)
        return 0
    lax.fori_loop(0, CH, _zero_row, 0, unroll=4)
    for i in range(rps // zb):
        pltpu.sync_copy(rows0_v.at[pl.ds(0, zb)],
                        acc_v.at[pl.ds(s * rps + i * zb, zb)])

    def _zero_d(i, _):
        for cc in range(D // LN):
            dacc_v[i, pl.ds(cc * LN, LN)] = jnp.zeros((LN,), jnp.float32)
        return 0
    lax.fori_loop(0, npad // D, _zero_d, 0, unroll=4)
    for i in range(npad // D // LN):
        idx80_v[pl.ds(i * LN, LN)] = i * LN + iota

    @pl.when(s == 0)
    def _():
        pltpu.sync_copy(rows0_v.at[pl.ds(0, npad // D)], dsp_v)
    plsc.subcore_barrier()

    def _stage(k):
        base = tbase + k * SEG
        pltpu.sync_copy(tgt_hbm.at[pl.ds(base, SEG)], tgt_s)
        pltpu.sync_copy(oth_hbm.at[pl.ds(base, SEG)], oth_s)
        pltpu.sync_copy(ef_hbm.at[pl.ds(base, SEG)], ef_s)

        def _sig_row(i, _):
            for cc in range(CH // LN):
                v = ef_s[i, pl.ds(cc * LN, LN)]
                ef_s[i, pl.ds(cc * LN, LN)] = 1.0 / (1.0 + jnp.exp(-v))
            return 0
        lax.fori_loop(0, SEG, _sig_row, 0, unroll=2)

    def _gstart(j, rbuf, sem):
        pltpu.async_copy(h2_hbm.at[oth_s.at[j]], rbuf, sem)

    def _gwait(rbuf, sem):
        pltpu.make_async_copy(h2_hbm.at[oth_s.at[0]], rbuf, sem).wait()

    def _sstart(j, rbuf, sem):
        pltpu.async_copy(rbuf, acc_v.at[tgt_s.at[j]], sem, add=True)

    def _swait(rbuf, sem):
        pltpu.make_async_copy(rbuf, acc_v.at[tgt_s.at[0]], sem).wait()

    def _work(j, rbuf):
        for g in range(CH // LN):
            wv = ef_s[j, pl.ds(g * LN, LN)]
            tv = tgt_s[j, pl.ds(g * LN, LN)]

            trow = lax.shift_right_logical(tv, 7)
            tcol = lax.bitwise_and(tv, jnp.int32(D - 1))
            for t in range(LN):
                plsc.addupdate_scatter(dacc_v, [trow, tcol], wv,
                                       mask=iota == t)

            def _scale_row(t, _, wv=wv, g=g, rbuf=rbuf):
                wr = _lane_gather(wv, jnp.full((LN,), t, jnp.int32))
                r = g * LN + t
                for cc in range(D // LN):
                    rbuf[r, pl.ds(cc * LN, LN)] = (
                        rbuf[r, pl.ds(cc * LN, LN)] * wr)
                return 0
            lax.fori_loop(0, LN, _scale_row, 0, unroll=2)

    _stage(0)
    _sstart(0, rows1_v, s1)
    _gstart(0, rows0_v, g0)

    def _segment(k, _):
        def _pair(i, _):
            j0 = 2 * i
            j1 = j0 + 1
            _swait(rows1_v, s1)
            _gstart(j1, rows1_v, g1)
            _gwait(rows0_v, g0)
            _work(j0, rows0_v)
            _sstart(j0, rows0_v, s0)
            _gwait(rows1_v, g1)
            _swait(rows0_v, s0)
            _gstart(j0 + 2, rows0_v, g0)
            _work(j1, rows1_v)
            _sstart(j1, rows1_v, s1)
            return 0
        lax.fori_loop(0, SEG // 2 - 1, _pair, 0)

        j0 = SEG - 2
        j1 = SEG - 1
        _swait(rows1_v, s1)
        _gstart(j1, rows1_v, g1)
        _gwait(rows0_v, g0)
        _work(j0, rows0_v)
        _sstart(j0, rows0_v, s0)
        _gwait(rows1_v, g1)
        _swait(rows0_v, s0)
        _work(j1, rows1_v)
        for cc in range(CH // LN):
            tlast_v[pl.ds(cc * LN, LN)] = tgt_s[j1, pl.ds(cc * LN, LN)]
        pltpu.async_copy(rows1_v, acc_v.at[tlast_v], s1, add=True)

        kn = jnp.minimum(k + 1, nseg - 1)
        _stage(kn)
        _gstart(0, rows0_v, g0)
        return 0
    lax.fori_loop(0, nseg, _segment, 0)

    _gwait(rows0_v, g0)
    _swait(rows1_v, s1)

    pltpu.sync_copy(dacc_v, dsp_v.at[idx80_v], add=True)

    plsc.subcore_barrier()

    @pl.when(s == 0)
    def _():
        pltpu.sync_copy(dsp_v, den_hbm.at[c])
    for i in range(rps // zb):
        r0 = s * rps + i * zb
        pltpu.sync_copy(acc_v.at[pl.ds(r0, zb)], msg_hbm.at[c, pl.ds(r0, zb)])


def kernel(node_features, edge_index, edge_features, global_features,
           W1a, b1a, W2a, b2a, W1b, b1b, W2b, b2b, W1c, b1c, W2c, b2c):
    x = node_features[0]
    n, d = x.shape
    e = edge_index.shape[2]
    gf = global_features[0]
    hdim = W1a.shape[0]

    npad = -(-n // (NSUB * D)) * NSUB * D
    xp = jnp.pad(x, ((0, npad - n), (0, 0)))
    grid = npad // BN

    dense = pl.pallas_call(
        _dense_body,
        grid=(grid,),
        in_specs=[
            pl.BlockSpec((BN, d), lambda i: (i, 0)),
            pl.BlockSpec((1, d), lambda i: (0, 0)),
            pl.BlockSpec((hdim, d), lambda i: (0, 0)),
            pl.BlockSpec((1, hdim), lambda i: (0, 0)),
            pl.BlockSpec((d, hdim), lambda i: (0, 0)),
            pl.BlockSpec((1, d), lambda i: (0, 0)),
            pl.BlockSpec((hdim, d), lambda i: (0, 0)),
            pl.BlockSpec((1, hdim), lambda i: (0, 0)),
            pl.BlockSpec((d, hdim), lambda i: (0, 0)),
            pl.BlockSpec((1, d), lambda i: (0, 0)),
            pl.BlockSpec((hdim, d), lambda i: (0, 0)),
            pl.BlockSpec((1, hdim), lambda i: (0, 0)),
            pl.BlockSpec((d, hdim), lambda i: (0, 0)),
            pl.BlockSpec((1, d), lambda i: (0, 0)),
        ],
        out_specs=[
            pl.BlockSpec((BN, d), lambda i: (i, 0)),
            pl.BlockSpec((BN, d), lambda i: (i, 0)),
            pl.BlockSpec((1, d), lambda i: (0, 0)),
        ],
        out_shape=[
            jax.ShapeDtypeStruct((npad, d), jnp.float32),
            jax.ShapeDtypeStruct((npad, d), jnp.float32),
            jax.ShapeDtypeStruct((1, d), jnp.float32),
        ],
    )
    h1, h2, g = dense(
        xp, gf,
        W1a, b1a.reshape(1, hdim), W2a, b2a.reshape(1, d),
        W1b, b1b.reshape(1, hdim), W2b, b2b.reshape(1, d),
        W1c, b1c.reshape(1, hdim), W2c, b2c.reshape(1, d))

    src = edge_index[0, 0]
    dst = edge_index[0, 1]
    ef = edge_features[0]
    tgt = jnp.concatenate([src, dst])
    oth = jnp.concatenate([dst, src])
    ef2 = jnp.concatenate([ef, ef])
    cpt = -(-2 * e // (NW * CH * SEG)) * SEG
    cpt0 = 304
    cpt1 = 2 * cpt - cpt0
    ep = NSUB * CH * (cpt0 + cpt1)
    pad = ep - 2 * e
    tgt = jnp.pad(tgt, (0, pad)).reshape(ep // CH, CH)
    oth = jnp.pad(oth, (0, pad)).reshape(ep // CH, CH)
    ef2 = jnp.pad(ef2, (0, pad), constant_values=-1e9).reshape(ep // CH, CH)

    mesh = plsc.VectorSubcoreMesh(core_axis_name="c", subcore_axis_name="s",
                                  num_cores=NCORE)
    edge_call = pl.kernel(
        functools.partial(_edge_body, npad, cpt0, cpt1),
        out_type=[
            pltpu.HBM((NCORE, npad, D), jnp.float32),
            pltpu.HBM((NCORE, npad // D, D), jnp.float32),
        ],
        mesh=mesh,
        scratch_types=[
            pltpu.VMEM((SEG, CH), jnp.int32),
            pltpu.VMEM((SEG, CH), jnp.int32),
            pltpu.VMEM((SEG, CH), jnp.float32),
            pltpu.VMEM((CH, D), jnp.float32),
            pltpu.VMEM((CH, D), jnp.float32),
            pltpu.VMEM((CH,), jnp.int32),
            pltpu.VMEM((npad // D, D), jnp.float32),
            pltpu.VMEM((npad // D,), jnp.int32),
            pltpu.VMEM_SHARED((npad, D), jnp.float32),
            pltpu.VMEM_SHARED((npad // D, D), jnp.float32),
            pltpu.SemaphoreType.DMA,
            pltpu.SemaphoreType.DMA,
            pltpu.SemaphoreType.DMA,
            pltpu.SemaphoreType.DMA,
        ],
        compiler_params=pltpu.CompilerParams(needs_layout_passes=False),
    )
    msgs, dens = edge_call(h2, tgt, oth, ef2)

    combine = pl.pallas_call(
        _combine_body,
        grid=(grid,),
        in_specs=[
            pl.BlockSpec((BN, d), lambda i: (i, 0)),
            pl.BlockSpec((BN, d), lambda i: (i, 0)),
            pl.BlockSpec((1, BN, d), lambda i: (0, i, 0)),
            pl.BlockSpec((1, BN, d), lambda i: (min(NCORE - 1, 1), i, 0)),
            pl.BlockSpec((NCORE, BN // D, D), lambda i: (0, i, 0)),
            pl.BlockSpec((1, d), lambda i: (0, 0)),
        ],
        out_specs=pl.BlockSpec((BN, d), lambda i: (i, 0)),
        out_shape=jax.ShapeDtypeStruct((npad, d), jnp.float32),
    )
    if NCORE == 1:
        out = combine(xp, h1, msgs, jnp.zeros_like(msgs), dens, g)
    else:
        out = combine(xp, h1, msgs, msgs, dens, g)
    return out[:n][None]


# speedup vs baseline: 1.0577x; 1.0577x over previous
import functools

import jax
import jax.numpy as jnp
from jax import lax
from jax.experimental import pallas as pl
from jax.experimental.pallas import tpu as pltpu
from jax.experimental.pallas import tpu_sc as plsc

D = 128
CH = 80
SEG = 16
LN = 16
NCORE = 2
NSUB = 16
NW = NCORE * NSUB
BN = 1024


def _fcnn_block(v, W1, b1, W2, b2):
    h = lax.dot_general(v, W1, (((1,), (1,)), ((), ())),
                        preferred_element_type=jnp.float32)
    h = jnp.maximum(h + b1, 0.0)
    o = lax.dot_general(h, W2, (((1,), (1,)), ((), ())),
                        preferred_element_type=jnp.float32)
    return o + b2


def _dense_body(x_ref, gf_ref, W1a_ref, b1a_ref, W2a_ref, b2a_ref,
                W1b_ref, b1b_ref, W2b_ref, b2b_ref,
                W1c_ref, b1c_ref, W2c_ref, b2c_ref,
                h1_ref, h2_ref, g_ref):
    x = x_ref[...]
    h1_ref[...] = _fcnn_block(x, W1a_ref[...], b1a_ref[...],
                              W2a_ref[...], b2a_ref[...])
    h2_ref[...] = _fcnn_block(x, W1b_ref[...], b1b_ref[...],
                              W2b_ref[...], b2b_ref[...])

    @pl.when(pl.program_id(0) == 0)
    def _():
        g_ref[...] = _fcnn_block(gf_ref[...], W1c_ref[...], b1c_ref[...],
                                 W2c_ref[...], b2c_ref[...])


def _combine_body(x_ref, h1_ref, a0_ref, a1_ref, d_ref, g_ref,
                  out_ref):
    msg = a0_ref[0] + a1_ref[0]
    bn = msg.shape[0]
    nr = bn // D
    dsum = jnp.sum(d_ref[...], axis=0)
    rowsel = (lax.broadcasted_iota(jnp.int32, (bn, nr), 0) // D
              == lax.broadcasted_iota(jnp.int32, (bn, nr), 1))
    dfull = lax.dot_general(rowsel.astype(jnp.float32), dsum,
                            (((1,), (0,)), ((), ())),
                            preferred_element_type=jnp.float32)
    lanesel = (lax.broadcasted_iota(jnp.int32, (bn, D), 0) % D
               == lax.broadcasted_iota(jnp.int32, (bn, D), 1))
    den = jnp.sum(jnp.where(lanesel, dfull, 0.0), axis=1,
                  keepdims=True) + jnp.float32(1e-7)
    inter = h1_ref[...] + msg / den + g_ref[...]
    m = jnp.mean(inter, axis=1, keepdims=True)
    cen = inter - m
    var = jnp.mean(cen * cen, axis=1, keepdims=True)
    normed = cen * lax.rsqrt(var + jnp.float32(1e-5))
    out_ref[...] = x_ref[...] + jnp.maximum(normed, 0.0)


def _lane_gather(vec, idx):
    return lax.gather(
        vec, idx[:, None],
        lax.GatherDimensionNumbers(offset_dims=(), collapsed_slice_dims=(0,),
                                   start_index_map=(0,)),
        (1,), mode=lax.GatherScatterMode.PROMISE_IN_BOUNDS)


def _edge_body(npad, cpt0, cpt1, h2_hbm, tgt_hbm, oth_hbm, ef_hbm,
               msg_hbm, den_hbm,
               tgt_s, oth_s, ef_s, rows0_v, rows1_v, tlast_v, dacc_v, idx80_v,
               acc_v, dsp_v, g0, g1, s0, s1):
    c = lax.axis_index("c")
    s = lax.axis_index("s")
    rps = npad // NSUB
    zb = 64
    nseg = jnp.where(c == 0, cpt0 // SEG, cpt1 // SEG)
    tbase = jnp.where(c == 0, s * cpt0, NSUB * cpt0 + s * cpt1)
    iota = lax.iota(jnp.int32, LN)

    def _zero_row(i, _):
        for cc in range(D // LN):
            rows0_v[i, pl.ds(cc * LN, LN)] = jnp.zeros((LN,), jnp.float32)
            rows1_v[i, pl.ds(cc * LN, LN)] = jnp.zeros((LN,), jnp.float32)
        return 0
    lax.fori_loop(0, CH, _zero_row, 0, unroll=4)
    for i in range(rps // zb):
        pltpu.sync_copy(rows0_v.at[pl.ds(0, zb)],
                        acc_v.at[pl.ds(s * rps + i * zb, zb)])

    def _zero_d(i, _):
        for cc in range(D // LN):
            dacc_v[i, pl.ds(cc * LN, LN)] = jnp.zeros((LN,), jnp.float32)
        return 0
    lax.fori_loop(0, npad // D, _zero_d, 0, unroll=4)
    for i in range(npad // D // LN):
        idx80_v[pl.ds(i * LN, LN)] = i * LN + iota

    @pl.when(s == 0)
    def _():
        pltpu.sync_copy(rows0_v.at[pl.ds(0, npad // D)], dsp_v)
    plsc.subcore_barrier()

    def _stage(k):
        base = tbase + k * SEG
        pltpu.sync_copy(tgt_hbm.at[pl.ds(base, SEG)], tgt_s)
        pltpu.sync_copy(oth_hbm.at[pl.ds(base, SEG)], oth_s)
        pltpu.sync_copy(ef_hbm.at[pl.ds(base, SEG)], ef_s)

        def _sig_row(i, _):
            for cc in range(CH // LN):
                v = ef_s[i, pl.ds(cc * LN, LN)]
                ef_s[i, pl.ds(cc * LN, LN)] = 1.0 / (1.0 + jnp.exp(-v))
            return 0
        lax.fori_loop(0, SEG, _sig_row, 0, unroll=2)

    def _gstart(j, rbuf, sem):
        pltpu.async_copy(h2_hbm.at[oth_s.at[j]], rbuf, sem)

    def _gwait(rbuf, sem):
        pltpu.make_async_copy(h2_hbm.at[oth_s.at[0]], rbuf, sem).wait()

    def _sstart(j, rbuf, sem):
        pltpu.async_copy(rbuf, acc_v.at[tgt_s.at[j]], sem, add=True)

    def _swait(rbuf, sem):
        pltpu.make_async_copy(rbuf, acc_v.at[tgt_s.at[0]], sem).wait()

    def _work(j, rbuf):
        for g in range(CH // LN):
            wv = ef_s[j, pl.ds(g * LN, LN)]
            tv = tgt_s[j, pl.ds(g * LN, LN)]

            trow = lax.shift_right_logical(tv, 7)
            tcol = lax.bitwise_and(tv, jnp.int32(D - 1))
            for t in range(LN):
                plsc.addupdate_scatter(dacc_v, [trow, tcol], wv,
                                       mask=iota == t)

            def _scale_row(t, _, wv=wv, g=g, rbuf=rbuf):
                wr = _lane_gather(wv, jnp.full((LN,), t, jnp.int32))
                r = g * LN + t
                for cc in range(D // LN):
                    rbuf[r, pl.ds(cc * LN, LN)] = (
                        rbuf[r, pl.ds(cc * LN, LN)] * wr)
                return 0
            lax.fori_loop(0, LN, _scale_row, 0, unroll=2)

    _stage(0)
    _sstart(0, rows1_v, s1)
    _gstart(0, rows0_v, g0)

    def _segment(k, _):
        def _pair(i, _):
            j0 = 2 * i
            j1 = j0 + 1
            _swait(rows1_v, s1)
            _gstart(j1, rows1_v, g1)
            _gwait(rows0_v, g0)
            _work(j0, rows0_v)
            _sstart(j0, rows0_v, s0)
            _gwait(rows1_v, g1)
            _swait(rows0_v, s0)
            _gstart(j0 + 2, rows0_v, g0)
            _work(j1, rows1_v)
            _sstart(j1, rows1_v, s1)
            return 0
        lax.fori_loop(0, SEG // 2 - 1, _pair, 0)

        j0 = SEG - 2
        j1 = SEG - 1
        _swait(rows1_v, s1)
        _gstart(j1, rows1_v, g1)
        _gwait(rows0_v, g0)
        _work(j0, rows0_v)
        _sstart(j0, rows0_v, s0)
        _gwait(rows1_v, g1)
        _swait(rows0_v, s0)
        _work(j1, rows1_v)
        for cc in range(CH // LN):
            tlast_v[pl.ds(cc * LN, LN)] = tgt_s[j1, pl.ds(cc * LN, LN)]
        pltpu.async_copy(rows1_v, acc_v.at[tlast_v], s1, add=True)

        kn = jnp.minimum(k + 1, nseg - 1)
        _stage(kn)
        _gstart(0, rows0_v, g0)
        return 0
    lax.fori_loop(0, nseg, _segment, 0)

    _gwait(rows0_v, g0)
    _swait(rows1_v, s1)

    pltpu.sync_copy(dacc_v, dsp_v.at[idx80_v], add=True)

    plsc.subcore_barrier()

    @pl.when(s == 0)
    def _():
        pltpu.sync_copy(dsp_v, den_hbm.at[c])
    for i in range(rps // zb):
        r0 = s * rps + i * zb
        pltpu.sync_copy(acc_v.at[pl.ds(r0, zb)], msg_hbm.at[c, pl.ds(r0, zb)])


def kernel(node_features, edge_index, edge_features, global_features,
           W1a, b1a, W2a, b2a, W1b, b1b, W2b, b2b, W1c, b1c, W2c, b2c):
    x = node_features[0]
    n, d = x.shape
    e = edge_index.shape[2]
    gf = global_features[0]
    hdim = W1a.shape[0]

    npad = -(-n // (NSUB * D)) * NSUB * D
    xp = jnp.pad(x, ((0, npad - n), (0, 0)))
    grid = npad // BN

    dense = pl.pallas_call(
        _dense_body,
        grid=(grid,),
        in_specs=[
            pl.BlockSpec((BN, d), lambda i: (i, 0)),
            pl.BlockSpec((1, d), lambda i: (0, 0)),
            pl.BlockSpec((hdim, d), lambda i: (0, 0)),
            pl.BlockSpec((1, hdim), lambda i: (0, 0)),
            pl.BlockSpec((d, hdim), lambda i: (0, 0)),
            pl.BlockSpec((1, d), lambda i: (0, 0)),
            pl.BlockSpec((hdim, d), lambda i: (0, 0)),
            pl.BlockSpec((1, hdim), lambda i: (0, 0)),
            pl.BlockSpec((d, hdim), lambda i: (0, 0)),
            pl.BlockSpec((1, d), lambda i: (0, 0)),
            pl.BlockSpec((hdim, d), lambda i: (0, 0)),
            pl.BlockSpec((1, hdim), lambda i: (0, 0)),
            pl.BlockSpec((d, hdim), lambda i: (0, 0)),
            pl.BlockSpec((1, d), lambda i: (0, 0)),
        ],
        out_specs=[
            pl.BlockSpec((BN, d), lambda i: (i, 0)),
            pl.BlockSpec((BN, d), lambda i: (i, 0)),
            pl.BlockSpec((1, d), lambda i: (0, 0)),
        ],
        out_shape=[
            jax.ShapeDtypeStruct((npad, d), jnp.float32),
            jax.ShapeDtypeStruct((npad, d), jnp.float32),
            jax.ShapeDtypeStruct((1, d), jnp.float32),
        ],
    )
    h1, h2, g = dense(
        xp, gf,
        W1a, b1a.reshape(1, hdim), W2a, b2a.reshape(1, d),
        W1b, b1b.reshape(1, hdim), W2b, b2b.reshape(1, d),
        W1c, b1c.reshape(1, hdim), W2c, b2c.reshape(1, d))

    src = edge_index[0, 0]
    dst = edge_index[0, 1]
    ef = edge_features[0]
    tgt = jnp.concatenate([src, dst])
    oth = jnp.concatenate([dst, src])
    ef2 = jnp.concatenate([ef, ef])
    cpt = -(-2 * e // (NW * CH * SEG)) * SEG
    cpt0 = 384
    cpt1 = 2 * cpt - cpt0
    ep = NSUB * CH * (cpt0 + cpt1)
    pad = ep - 2 * e
    tgt = jnp.pad(tgt, (0, pad)).reshape(ep // CH, CH)
    oth = jnp.pad(oth, (0, pad)).reshape(ep // CH, CH)
    ef2 = jnp.pad(ef2, (0, pad), constant_values=-1e9).reshape(ep // CH, CH)

    mesh = plsc.VectorSubcoreMesh(core_axis_name="c", subcore_axis_name="s",
                                  num_cores=NCORE)
    edge_call = pl.kernel(
        functools.partial(_edge_body, npad, cpt0, cpt1),
        out_type=[
            pltpu.HBM((NCORE, npad, D), jnp.float32),
            pltpu.HBM((NCORE, npad // D, D), jnp.float32),
        ],
        mesh=mesh,
        scratch_types=[
            pltpu.VMEM((SEG, CH), jnp.int32),
            pltpu.VMEM((SEG, CH), jnp.int32),
            pltpu.VMEM((SEG, CH), jnp.float32),
            pltpu.VMEM((CH, D), jnp.float32),
            pltpu.VMEM((CH, D), jnp.float32),
            pltpu.VMEM((CH,), jnp.int32),
            pltpu.VMEM((npad // D, D), jnp.float32),
            pltpu.VMEM((npad // D,), jnp.int32),
            pltpu.VMEM_SHARED((npad, D), jnp.float32),
            pltpu.VMEM_SHARED((npad // D, D), jnp.float32),
            pltpu.SemaphoreType.DMA,
            pltpu.SemaphoreType.DMA,
            pltpu.SemaphoreType.DMA,
            pltpu.SemaphoreType.DMA,
        ],
        compiler_params=pltpu.CompilerParams(needs_layout_passes=False),
    )
    msgs, dens = edge_call(h2, tgt, oth, ef2)

    combine = pl.pallas_call(
        _combine_body,
        grid=(grid,),
        in_specs=[
            pl.BlockSpec((BN, d), lambda i: (i, 0)),
            pl.BlockSpec((BN, d), lambda i: (i, 0)),
            pl.BlockSpec((1, BN, d), lambda i: (0, i, 0)),
            pl.BlockSpec((1, BN, d), lambda i: (min(NCORE - 1, 1), i, 0)),
            pl.BlockSpec((NCORE, BN // D, D), lambda i: (0, i, 0)),
            pl.BlockSpec((1, d), lambda i: (0, 0)),
        ],
        out_specs=pl.BlockSpec((BN, d), lambda i: (i, 0)),
        out_shape=jax.ShapeDtypeStruct((npad, d), jnp.float32),
    )
    if NCORE == 1:
        out = combine(xp, h1, msgs, jnp.zeros_like(msgs), dens, g)
    else:
        out = combine(xp, h1, msgs, msgs, dens, g)
    return out[:n][None]
